# baseline (speedup 1.0000x reference)
import jax
import jax.numpy as jnp
from jax import lax
from jax.experimental import pallas as pl
from jax.experimental.pallas import tpu as pltpu
from jax.experimental.pallas import tpu_sc as plsc

B, S, D = 4, 8192, 768
NC, NS = 2, 16
NW = NC * NS
ROWS_W = S // NW
R = 16
STEPS = ROWS_W // R
VPR = D // 16


def _sc_body(x_hbm, pe_hbm, out_hbm,
             xb00, xb01, xb10, xb11, xb20, xb21, xb30, xb31, peb0, peb1,
             si00, si01, si10, si11, si20, si21, si30, si31,
             so00, so01, so10, so11, so20, so21, so30, so31, sp0, sp1):
    wid = lax.axis_index("s") * NC + lax.axis_index("c")
    s0 = wid * ROWS_W
    xb = ((xb00, xb01), (xb10, xb11), (xb20, xb21), (xb30, xb31))
    sin = ((si00, si01), (si10, si11), (si20, si21), (si30, si31))
    sout = ((so00, so01), (so10, so11), (so20, so21), (so30, so31))
    pebufs = (peb0, peb1)
    spe = (sp0, sp1)

    def pe_src(c):
        return pe_hbm.at[pl.ds(s0 + c * R, R)]

    def x_src(c, b):
        return x_hbm.at[b, pl.ds(s0 + c * R, R)]

    def o_dst(c, b):
        return out_hbm.at[b, pl.ds(s0 + c * R, R)]

    pltpu.async_copy(pe_src(0), peb0, sp0)
    pltpu.async_copy(pe_src(1), peb1, sp1)
    for b in range(B):
        pltpu.async_copy(x_src(0, b), xb[b][0], sin[b][0])

    def body(ci, carry):
        for p in range(2):
            cj = ci * 2 + p
            peb = pebufs[p]
            pltpu.make_async_copy(pe_src(cj), peb, spe[p]).wait()
            for b in range(B):
                xbuf = xb[b][p]
                pltpu.make_async_copy(x_src(cj, b), xbuf, sin[b][p]).wait()

                def row_fn(i, cr, xbuf=xbuf, peb=peb):
                    for k in range(VPR):
                        q = k * 16
                        plsc.addupdate(xbuf.at[i, pl.ds(q, 16)],
                                       peb[i, pl.ds(q, 16)])
                    return cr

                lax.fori_loop(0, R, row_fn, 0)
                pltpu.async_copy(xbuf, o_dst(cj, b), sout[b][p])

                nxt = xb[b][1 - p]

                @pl.when(jnp.logical_and(cj >= 1, cj + 1 < STEPS))
                def _(b=b, p=p, nxt=nxt, cj=cj):
                    pltpu.make_async_copy(nxt, o_dst(cj, b),
                                          sout[b][1 - p]).wait()

                @pl.when(cj + 1 < STEPS)
                def _(b=b, p=p, nxt=nxt, cj=cj):
                    pltpu.async_copy(x_src(cj + 1, b), nxt, sin[b][1 - p])

            @pl.when(cj + 2 < STEPS)
            def _(peb=peb, p=p, cj=cj):
                pltpu.async_copy(pe_src(cj + 2), peb, spe[p])
        return carry

    lax.fori_loop(0, STEPS // 2, body, 0)

    for b in range(B):
        pltpu.make_async_copy(xb[b][0], o_dst(STEPS - 2, b), sout[b][0]).wait()
        pltpu.make_async_copy(xb[b][1], o_dst(STEPS - 1, b), sout[b][1]).wait()


def kernel(x, pe):
    mesh = plsc.VectorSubcoreMesh(
        core_axis_name="c", subcore_axis_name="s", num_cores=NC, num_subcores=NS
    )
    f = pl.kernel(
        _sc_body,
        out_type=jax.ShapeDtypeStruct((B, S, D), jnp.float32),
        mesh=mesh,
        scratch_types=(
            [pltpu.VMEM((R, D), jnp.float32)] * 10
            + [pltpu.SemaphoreType.DMA] * 18
        ),
    )
    return f(x, pe)


# speedup vs baseline: 1.0089x; 1.0089x over previous
import jax
import jax.numpy as jnp
from jax import lax
from jax.experimental import pallas as pl
from jax.experimental.pallas import tpu as pltpu
from jax.experimental.pallas import tpu_sc as plsc

B, S, D = 4, 8192, 768
NC, NS = 2, 16
NW = NC * NS
ROWS_W = S // NW
R = 16
STEPS = ROWS_W // R
VPR = D // 16


def _sc_body(x_hbm, pe_hbm, out_hbm,
             xb00, xb01, xb10, xb11, xb20, xb21, xb30, xb31, peb0, peb1,
             si00, si01, si10, si11, si20, si21, si30, si31,
             so00, so01, so10, so11, so20, so21, so30, so31, sp0, sp1):
    wid = lax.axis_index("s") * NC + lax.axis_index("c")
    s0 = wid * ROWS_W
    xb = ((xb00, xb01), (xb10, xb11), (xb20, xb21), (xb30, xb31))
    sin = ((si00, si01), (si10, si11), (si20, si21), (si30, si31))
    sout = ((so00, so01), (so10, so11), (so20, so21), (so30, so31))
    pebufs = (peb0, peb1)
    spe = (sp0, sp1)

    def pe_src(c):
        return pe_hbm.at[pl.ds(s0 + c * R, R)]

    def x_src(c, b):
        return x_hbm.at[b, pl.ds(s0 + c * R, R)]

    def o_dst(c, b):
        return out_hbm.at[b, pl.ds(s0 + c * R, R)]

    pltpu.async_copy(pe_src(0), peb0, sp0)
    pltpu.async_copy(pe_src(1), peb1, sp1)
    for b in range(B):
        pltpu.async_copy(x_src(0, b), xb[b][0], sin[b][0])

    def body(ci, carry):
        for p in range(2):
            cj = ci * 2 + p
            peb = pebufs[p]
            pltpu.make_async_copy(pe_src(cj), peb, spe[p]).wait()
            for b in range(B):
                xbuf = xb[b][p]
                pltpu.make_async_copy(x_src(cj, b), xbuf, sin[b][p]).wait()

                nxt = xb[b][1 - p]

                @pl.when(jnp.logical_and(cj >= 1, cj + 1 < STEPS))
                def _(b=b, p=p, nxt=nxt, cj=cj):
                    pltpu.make_async_copy(nxt, o_dst(cj, b),
                                          sout[b][1 - p]).wait()

                @pl.when(cj + 1 < STEPS)
                def _(b=b, p=p, nxt=nxt, cj=cj):
                    pltpu.async_copy(x_src(cj + 1, b), nxt, sin[b][1 - p])

                def row_fn(i, cr, xbuf=xbuf, peb=peb):
                    for k in range(VPR):
                        q = k * 16
                        plsc.addupdate(xbuf.at[i, pl.ds(q, 16)],
                                       peb[i, pl.ds(q, 16)])
                    return cr

                lax.fori_loop(0, R, row_fn, 0)
                pltpu.async_copy(xbuf, o_dst(cj, b), sout[b][p])

            @pl.when(cj + 2 < STEPS)
            def _(peb=peb, p=p, cj=cj):
                pltpu.async_copy(pe_src(cj + 2), peb, spe[p])
        return carry

    lax.fori_loop(0, STEPS // 2, body, 0)

    for b in range(B):
        pltpu.make_async_copy(xb[b][0], o_dst(STEPS - 2, b), sout[b][0]).wait()
        pltpu.make_async_copy(xb[b][1], o_dst(STEPS - 1, b), sout[b][1]).wait()


def kernel(x, pe):
    mesh = plsc.VectorSubcoreMesh(
        core_axis_name="c", subcore_axis_name="s", num_cores=NC, num_subcores=NS
    )
    f = pl.kernel(
        _sc_body,
        out_type=jax.ShapeDtypeStruct((B, S, D), jnp.float32),
        mesh=mesh,
        scratch_types=(
            [pltpu.VMEM((R, D), jnp.float32)] * 10
            + [pltpu.SemaphoreType.DMA] * 18
        ),
    )
    return f(x, pe)
